# Initial kernel scaffold; baseline (speedup 1.0000x reference)
import jax
import jax.numpy as jnp
from jax.experimental import pallas as pl


def kernel(x, w1, b1, w2, b2):
    raise NotImplementedError("write your pallas kernel here")


# speedup vs baseline: 1.4176x; 1.4176x over previous
import jax
import jax.numpy as jnp
from jax.experimental import pallas as pl
from jax.experimental.pallas import tpu as pltpu


def _mlp_kernel(x_ref, w1_ref, b1_ref, w2_ref, b2_ref, o_ref):
    h = jnp.dot(x_ref[...], w1_ref[...], preferred_element_type=jnp.float32)
    h = jnp.maximum(h + b1_ref[...], 0.0)
    y = jnp.dot(h, w2_ref[...], preferred_element_type=jnp.float32)
    o_ref[...] = (y + b2_ref[...]).astype(o_ref.dtype)


def _round_up(a, m):
    return ((a + m - 1) // m) * m


def kernel(x, w1, b1, w2, b2, *, block_b=2048):
    B, K = x.shape
    inter_dim = w1.shape[1]
    out_dim = w2.shape[1]

    block_b = max(8, min(block_b, _round_up(B, 8)))
    B_pad = _round_up(B, block_b)
    if B_pad != B:
        x = jnp.pad(x, ((0, B_pad - B), (0, 0)))
    num_blocks = B_pad // block_b

    out = pl.pallas_call(
        _mlp_kernel,
        out_shape=jax.ShapeDtypeStruct((B_pad, out_dim), x.dtype),
        grid=(num_blocks,),
        in_specs=[
            pl.BlockSpec((block_b, K), lambda i: (i, 0)),
            pl.BlockSpec((K, inter_dim), lambda i: (0, 0)),
            pl.BlockSpec((1, inter_dim), lambda i: (0, 0)),
            pl.BlockSpec((inter_dim, out_dim), lambda i: (0, 0)),
            pl.BlockSpec((1, out_dim), lambda i: (0, 0)),
        ],
        out_specs=pl.BlockSpec((block_b, out_dim), lambda i: (i, 0)),
        compiler_params=pltpu.CompilerParams(
            dimension_semantics=("parallel",),
            vmem_limit_bytes=64 * 1024 * 1024,
        ),
    )(x, w1, b1, w2, b2)

    if B_pad != B:
        out = out[:B]
    return out


# speedup vs baseline: 1.7718x; 1.2498x over previous
import jax
import jax.numpy as jnp
from jax.experimental import pallas as pl
from jax.experimental.pallas import tpu as pltpu


def _mlp_kernel(x_ref, w1_ref, b1_ref, w2_ref, b2_ref, o_ref):
    h = jnp.dot(x_ref[...], w1_ref[...], preferred_element_type=jnp.float32)
    h = jnp.maximum(h + b1_ref[...], 0.0)
    y = jnp.dot(h, w2_ref[...], preferred_element_type=jnp.float32)
    o_ref[...] = (y + b2_ref[...]).astype(o_ref.dtype)


def _round_up(a, m):
    return ((a + m - 1) // m) * m


def kernel(x, w1, b1, w2, b2, *, block_b=8192):
    B, K = x.shape
    inter_dim = w1.shape[1]
    out_dim = w2.shape[1]

    block_b = max(8, min(block_b, _round_up(B, 8)))
    B_pad = _round_up(B, block_b)
    if B_pad != B:
        x = jnp.pad(x, ((0, B_pad - B), (0, 0)))
    num_blocks = B_pad // block_b

    out = pl.pallas_call(
        _mlp_kernel,
        out_shape=jax.ShapeDtypeStruct((B_pad, out_dim), x.dtype),
        grid=(num_blocks,),
        in_specs=[
            pl.BlockSpec((block_b, K), lambda i: (i, 0)),
            pl.BlockSpec((K, inter_dim), lambda i: (0, 0)),
            pl.BlockSpec((1, inter_dim), lambda i: (0, 0)),
            pl.BlockSpec((inter_dim, out_dim), lambda i: (0, 0)),
            pl.BlockSpec((1, out_dim), lambda i: (0, 0)),
        ],
        out_specs=pl.BlockSpec((block_b, out_dim), lambda i: (i, 0)),
        compiler_params=pltpu.CompilerParams(
            dimension_semantics=("parallel",),
            vmem_limit_bytes=64 * 1024 * 1024,
        ),
    )(x, w1, b1, w2, b2)

    if B_pad != B:
        out = out[:B]
    return out


# speedup vs baseline: 1.7851x; 1.0075x over previous
import jax
import jax.numpy as jnp
from jax.experimental import pallas as pl
from jax.experimental.pallas import tpu as pltpu


def _mlp_kernel(x_ref, w1_ref, b1_ref, w2_ref, b2_ref, o_ref):
    h = jnp.dot(x_ref[...], w1_ref[...], preferred_element_type=jnp.float32)
    h = jnp.maximum(h + b1_ref[...], 0.0)
    y = jnp.dot(h, w2_ref[...], preferred_element_type=jnp.float32)
    o_ref[...] = (y + b2_ref[...]).astype(o_ref.dtype)


def _round_up(a, m):
    return ((a + m - 1) // m) * m


def kernel(x, w1, b1, w2, b2, *, block_b=16384):
    B, K = x.shape
    inter_dim = w1.shape[1]
    out_dim = w2.shape[1]

    block_b = max(8, min(block_b, _round_up(B, 8)))
    B_pad = _round_up(B, block_b)
    if B_pad != B:
        x = jnp.pad(x, ((0, B_pad - B), (0, 0)))
    num_blocks = B_pad // block_b

    out = pl.pallas_call(
        _mlp_kernel,
        out_shape=jax.ShapeDtypeStruct((B_pad, out_dim), x.dtype),
        grid=(num_blocks,),
        in_specs=[
            pl.BlockSpec((block_b, K), lambda i: (i, 0)),
            pl.BlockSpec((K, inter_dim), lambda i: (0, 0)),
            pl.BlockSpec((1, inter_dim), lambda i: (0, 0)),
            pl.BlockSpec((inter_dim, out_dim), lambda i: (0, 0)),
            pl.BlockSpec((1, out_dim), lambda i: (0, 0)),
        ],
        out_specs=pl.BlockSpec((block_b, out_dim), lambda i: (i, 0)),
        compiler_params=pltpu.CompilerParams(
            dimension_semantics=("parallel",),
            vmem_limit_bytes=64 * 1024 * 1024,
        ),
    )(x, w1, b1, w2, b2)

    if B_pad != B:
        out = out[:B]
    return out


# speedup vs baseline: 12.3238x; 6.9038x over previous
import jax
import jax.numpy as jnp
from jax.experimental import pallas as pl
from jax.experimental.pallas import tpu as pltpu


def _mlp_t_kernel(xt_ref, w1t_ref, b1t_ref, w2t_ref, b2t_ref, ot_ref):
    h = jnp.dot(w1t_ref[...], xt_ref[...], preferred_element_type=jnp.float32)
    h = jnp.maximum(h + b1t_ref[...], 0.0)
    y = jnp.dot(w2t_ref[...], h, preferred_element_type=jnp.float32)
    ot_ref[...] = (y + b2t_ref[...]).astype(ot_ref.dtype)


def _round_up(a, m):
    return ((a + m - 1) // m) * m


def kernel(x, w1, b1, w2, b2, *, block_b=16384):
    B, K = x.shape
    inter_dim = w1.shape[1]
    out_dim = w2.shape[1]

    block_b = max(128, min(block_b, _round_up(B, 128)))
    B_pad = _round_up(B, block_b)
    if B_pad != B:
        x = jnp.pad(x, ((0, B_pad - B), (0, 0)))
    num_blocks = B_pad // block_b

    xt = x.T
    w1t = w1.T
    w2t = w2.T
    b1t = b1.T
    b2t = b2.T

    out_t = pl.pallas_call(
        _mlp_t_kernel,
        out_shape=jax.ShapeDtypeStruct((out_dim, B_pad), x.dtype),
        grid=(num_blocks,),
        in_specs=[
            pl.BlockSpec((K, block_b), lambda i: (0, i)),
            pl.BlockSpec((inter_dim, K), lambda i: (0, 0)),
            pl.BlockSpec((inter_dim, 1), lambda i: (0, 0)),
            pl.BlockSpec((out_dim, inter_dim), lambda i: (0, 0)),
            pl.BlockSpec((out_dim, 1), lambda i: (0, 0)),
        ],
        out_specs=pl.BlockSpec((out_dim, block_b), lambda i: (0, i)),
        compiler_params=pltpu.CompilerParams(
            dimension_semantics=("parallel",),
            vmem_limit_bytes=64 * 1024 * 1024,
        ),
    )(xt, w1t, b1t, w2t, b2t)

    out = out_t.T
    if B_pad != B:
        out = out[:B]
    return out


# speedup vs baseline: 13.4889x; 1.0945x over previous
import jax
import jax.numpy as jnp
from jax.experimental import pallas as pl
from jax.experimental.pallas import tpu as pltpu


def _mlp_t_kernel(xt_ref, w1t_ref, b1t_ref, w2t_ref, b2t_ref, ot_ref):
    h = jnp.dot(w1t_ref[...], xt_ref[...], preferred_element_type=jnp.float32)
    h = jnp.maximum(h + b1t_ref[...], 0.0)
    y = jnp.dot(w2t_ref[...], h, preferred_element_type=jnp.float32)
    ot_ref[...] = (y + b2t_ref[...]).astype(ot_ref.dtype)


def _round_up(a, m):
    return ((a + m - 1) // m) * m


def kernel(x, w1, b1, w2, b2, *, block_b=32768):
    B, K = x.shape
    inter_dim = w1.shape[1]
    out_dim = w2.shape[1]

    block_b = max(128, min(block_b, _round_up(B, 128)))
    B_pad = _round_up(B, block_b)
    if B_pad != B:
        x = jnp.pad(x, ((0, B_pad - B), (0, 0)))
    num_blocks = B_pad // block_b

    xt = x.T
    w1t = w1.T
    w2t = w2.T
    b1t = b1.T
    b2t = b2.T

    out_t = pl.pallas_call(
        _mlp_t_kernel,
        out_shape=jax.ShapeDtypeStruct((out_dim, B_pad), x.dtype),
        grid=(num_blocks,),
        in_specs=[
            pl.BlockSpec((K, block_b), lambda i: (0, i)),
            pl.BlockSpec((inter_dim, K), lambda i: (0, 0)),
            pl.BlockSpec((inter_dim, 1), lambda i: (0, 0)),
            pl.BlockSpec((out_dim, inter_dim), lambda i: (0, 0)),
            pl.BlockSpec((out_dim, 1), lambda i: (0, 0)),
        ],
        out_specs=pl.BlockSpec((out_dim, block_b), lambda i: (0, i)),
        compiler_params=pltpu.CompilerParams(
            dimension_semantics=("parallel",),
            vmem_limit_bytes=64 * 1024 * 1024,
        ),
    )(xt, w1t, b1t, w2t, b2t)

    out = out_t.T
    if B_pad != B:
        out = out[:B]
    return out


# speedup vs baseline: 13.6636x; 1.0130x over previous
import jax
import jax.numpy as jnp
from jax.experimental import pallas as pl
from jax.experimental.pallas import tpu as pltpu


def _mlp_t_kernel(xt_ref, w1t_ref, b1t_ref, w2t_ref, b2t_ref, ot_ref):
    h = jnp.dot(w1t_ref[...], xt_ref[...], preferred_element_type=jnp.float32)
    h = jnp.maximum(h + b1t_ref[...], 0.0)
    y = jnp.dot(w2t_ref[...], h, preferred_element_type=jnp.float32)
    ot_ref[...] = (y + b2t_ref[...]).astype(ot_ref.dtype)


def _round_up(a, m):
    return ((a + m - 1) // m) * m


def kernel(x, w1, b1, w2, b2, *, block_b=65536):
    B, K = x.shape
    inter_dim = w1.shape[1]
    out_dim = w2.shape[1]

    block_b = max(128, min(block_b, _round_up(B, 128)))
    B_pad = _round_up(B, block_b)
    if B_pad != B:
        x = jnp.pad(x, ((0, B_pad - B), (0, 0)))
    num_blocks = B_pad // block_b

    xt = x.T
    w1t = w1.T
    w2t = w2.T
    b1t = b1.T
    b2t = b2.T

    out_t = pl.pallas_call(
        _mlp_t_kernel,
        out_shape=jax.ShapeDtypeStruct((out_dim, B_pad), x.dtype),
        grid=(num_blocks,),
        in_specs=[
            pl.BlockSpec((K, block_b), lambda i: (0, i)),
            pl.BlockSpec((inter_dim, K), lambda i: (0, 0)),
            pl.BlockSpec((inter_dim, 1), lambda i: (0, 0)),
            pl.BlockSpec((out_dim, inter_dim), lambda i: (0, 0)),
            pl.BlockSpec((out_dim, 1), lambda i: (0, 0)),
        ],
        out_specs=pl.BlockSpec((out_dim, block_b), lambda i: (0, i)),
        compiler_params=pltpu.CompilerParams(
            dimension_semantics=("parallel",),
            vmem_limit_bytes=64 * 1024 * 1024,
        ),
    )(xt, w1t, b1t, w2t, b2t)

    out = out_t.T
    if B_pad != B:
        out = out[:B]
    return out


# speedup vs baseline: 14.1582x; 1.0362x over previous
import jax
import jax.numpy as jnp
from jax.experimental import pallas as pl
from jax.experimental.pallas import tpu as pltpu


def _mlp_t_kernel(xt_ref, w1t_ref, p_ref, ot_ref):
    inter = w1t_ref.shape[0]
    out_dim = ot_ref.shape[0]
    w2t = p_ref[:out_dim, :inter]
    b1t = p_ref[:inter, inter:inter + 1]
    b2t = p_ref[:out_dim, inter + 1:inter + 2]
    h = jnp.dot(w1t_ref[...], xt_ref[...], preferred_element_type=jnp.float32)
    h = jnp.maximum(h + b1t, 0.0)
    y = jnp.dot(w2t, h, preferred_element_type=jnp.float32)
    ot_ref[...] = (y + b2t).astype(ot_ref.dtype)


def _round_up(a, m):
    return ((a + m - 1) // m) * m


def kernel(x, w1, b1, w2, b2, *, block_b=65536):
    B, K = x.shape
    inter_dim = w1.shape[1]
    out_dim = w2.shape[1]

    block_b = max(128, min(block_b, _round_up(B, 128)))
    B_pad = _round_up(B, block_b)
    if B_pad != B:
        x = jnp.pad(x, ((0, B_pad - B), (0, 0)))
    num_blocks = B_pad // block_b

    xt = x.T
    w1t = w1.T
    rows = max(inter_dim, out_dim)
    packed = jnp.concatenate(
        [jnp.pad(a, ((0, rows - a.shape[0]), (0, 0)))
         for a in (w2.T, b1.T, b2.T)], axis=1)

    out_t = pl.pallas_call(
        _mlp_t_kernel,
        out_shape=jax.ShapeDtypeStruct((out_dim, B_pad), x.dtype),
        grid=(num_blocks,),
        in_specs=[
            pl.BlockSpec((K, block_b), lambda i: (0, i)),
            pl.BlockSpec((inter_dim, K), lambda i: (0, 0)),
            pl.BlockSpec((max(inter_dim, out_dim), inter_dim + 2),
                         lambda i: (0, 0)),
        ],
        out_specs=pl.BlockSpec((out_dim, block_b), lambda i: (0, i)),
        compiler_params=pltpu.CompilerParams(
            dimension_semantics=("parallel",),
            vmem_limit_bytes=64 * 1024 * 1024,
        ),
    )(xt, w1t, packed)

    out = out_t.T
    if B_pad != B:
        out = out[:B]
    return out


# speedup vs baseline: 15.5461x; 1.0980x over previous
import jax
import jax.numpy as jnp
from jax.experimental import pallas as pl
from jax.experimental.pallas import tpu as pltpu


def _mlp_t_kernel(xt_ref, w1t_ref, b1_ref, w2_ref, b2_ref, ot_ref):
    b1t = b1_ref[...].T
    b2t = b2_ref[...].T
    h = jnp.dot(w1t_ref[...], xt_ref[...], preferred_element_type=jnp.float32)
    h = jnp.maximum(h + b1t, 0.0)
    y = jax.lax.dot_general(w2_ref[...], h, (((0,), (0,)), ((), ())),
                            preferred_element_type=jnp.float32)
    ot_ref[...] = (y + b2t).astype(ot_ref.dtype)


def _round_up(a, m):
    return ((a + m - 1) // m) * m


def kernel(x, w1, b1, w2, b2, *, block_b=65536):
    B, K = x.shape
    inter_dim = w1.shape[1]
    out_dim = w2.shape[1]

    block_b = max(128, min(block_b, _round_up(B, 128)))
    B_pad = _round_up(B, block_b)
    if B_pad != B:
        x = jnp.pad(x, ((0, B_pad - B), (0, 0)))
    num_blocks = B_pad // block_b

    xt = x.T
    w1t = w1.T

    out_t = pl.pallas_call(
        _mlp_t_kernel,
        out_shape=jax.ShapeDtypeStruct((out_dim, B_pad), x.dtype),
        grid=(num_blocks,),
        in_specs=[
            pl.BlockSpec((K, block_b), lambda i: (0, i)),
            pl.BlockSpec((inter_dim, K), lambda i: (0, 0)),
            pl.BlockSpec((1, inter_dim), lambda i: (0, 0)),
            pl.BlockSpec((inter_dim, out_dim), lambda i: (0, 0)),
            pl.BlockSpec((1, out_dim), lambda i: (0, 0)),
        ],
        out_specs=pl.BlockSpec((out_dim, block_b), lambda i: (0, i)),
        compiler_params=pltpu.CompilerParams(
            dimension_semantics=("parallel",),
            vmem_limit_bytes=64 * 1024 * 1024,
        ),
    )(xt, w1t, b1, w2, b2)

    out = out_t.T
    if B_pad != B:
        out = out[:B]
    return out
